# Initial kernel scaffold; baseline (speedup 1.0000x reference)
import functools

import jax
import jax.numpy as jnp
from jax import lax
from jax.experimental import pallas as pl
from jax.experimental.pallas import tpu as pltpu
from jax.experimental.pallas import tpu_sc as plsc

N = 10000
E = 160000
D = 256
H = 128
NC = 2
NS = 16
L = 16
EPT = E // NS
SP = N // NS
BK = 128
NB = EPT // BK
TAIL = EPT - NB * BK


def _sc_propagate(x_stack, rowi, coli):
    mesh = plsc.VectorSubcoreMesh(
        core_axis_name="c", subcore_axis_name="s", num_cores=NC, num_subcores=NS
    )

    @functools.partial(
        pl.kernel,
        out_type=(
            jax.ShapeDtypeStruct((NC * N, H), jnp.float32),
            jax.ShapeDtypeStruct((N, L), jnp.float32),
            jax.ShapeDtypeStruct((NC * N, H), jnp.float32),
        ),
        mesh=mesh,
        scratch_types=(
            pltpu.VMEM_SHARED((N, H), jnp.float32),
            pltpu.VMEM_SHARED((N, L), jnp.float32),
            pltpu.VMEM((BK, L), jnp.float32),
            pltpu.VMEM((TAIL, L), jnp.float32),
            pltpu.VMEM((BK,), jnp.int32),
            pltpu.VMEM((TAIL,), jnp.int32),
            pltpu.VMEM((BK,), jnp.int32),
            pltpu.VMEM((TAIL,), jnp.int32),
            pltpu.VMEM((BK, H), jnp.float32),
            pltpu.VMEM((TAIL, H), jnp.float32),
            pltpu.VMEM((SP, L), jnp.float32),
            pltpu.VMEM((125, H), jnp.float32),
            pltpu.VMEM((125, H), jnp.float32),
            pltpu.VMEM((125, L), jnp.float32),
            pltpu.SemaphoreType.DMA,
        ),
    )
    def k(x_hbm, row_hbm, col_hbm, p_out, dis_out, y_hbm,
          p_sh, deg_sh, ones_b, ones_t, rowv, rowv_t, colv, colv_t,
          rowsb, rowsb_t, disv, xbuf, zbuf, zdeg, sem):
        cid = lax.axis_index("c")
        sid = lax.axis_index("s")
        nbase = sid * SP
        ebase = sid * EPT
        coff = cid * N

        zv = jnp.zeros((L,), jnp.float32)
        ov = jnp.ones((L,), jnp.float32)

        @pl.loop(0, 125)
        def _(r):
            zdeg[r, :] = zv
            for c in range(H // L):
                zbuf[r, pl.ds(c * L, L)] = zv

        @pl.loop(0, BK)
        def _(r):
            ones_b[r, :] = ov

        @pl.loop(0, TAIL)
        def _(r):
            ones_t[r, :] = ov

        for j in range(5):
            pltpu.sync_copy(zbuf, p_sh.at[pl.ds(nbase + j * 125, 125)])
            pltpu.sync_copy(zdeg, deg_sh.at[pl.ds(nbase + j * 125, 125)])
        plsc.subcore_barrier()

        @pl.loop(0, NB)
        def _(b):
            off = pl.multiple_of(ebase + b * BK, 8)
            pltpu.sync_copy(row_hbm.at[pl.ds(off, BK)], rowv)
            pltpu.sync_copy(ones_b, deg_sh.at[rowv], add=True)

        toff = pl.multiple_of(ebase + NB * BK, 8)
        pltpu.sync_copy(row_hbm.at[pl.ds(toff, TAIL)], rowv_t)
        pltpu.sync_copy(ones_t, deg_sh.at[rowv_t], add=True)
        plsc.subcore_barrier()

        pltpu.sync_copy(deg_sh.at[pl.ds(nbase, SP)], disv)

        @pl.loop(0, SP)
        def _(r):
            x = disv[r, :]
            i = lax.bitcast_convert_type(x, jnp.int32)
            y = lax.bitcast_convert_type(
                jnp.int32(0x5F3759DF) - (i >> 1), jnp.float32)
            for _ in range(3):
                y = y * (1.5 - 0.5 * x * y * y)
            disv[r, :] = jnp.where(x > 0.5, y, 0.0)

        @pl.when(cid == 0)
        def _():
            pltpu.sync_copy(disv, dis_out.at[pl.ds(nbase, SP)])

        for j in range(5):
            r0 = nbase + j * 125
            pltpu.sync_copy(x_hbm.at[cid, pl.ds(r0, 125)], xbuf)

            @pl.loop(0, 125)
            def _(rr):
                bc = lax.broadcast(disv[j * 125 + rr, 0], (L,))
                for c in range(H // L):
                    xbuf[rr, pl.ds(c * L, L)] = xbuf[rr, pl.ds(c * L, L)] * bc

            pltpu.sync_copy(xbuf, y_hbm.at[pl.ds(coff + r0, 125)])
        plsc.subcore_barrier()

        @pl.loop(0, NB)
        def _(b):
            off = pl.multiple_of(ebase + b * BK, 8)
            pltpu.sync_copy(col_hbm.at[pl.ds(off, BK)], colv)
            for c in range(BK // L):
                colv[pl.ds(c * L, L)] = colv[pl.ds(c * L, L)] + coff
            pltpu.async_copy(y_hbm.at[colv], rowsb, sem).wait()
            pltpu.sync_copy(row_hbm.at[pl.ds(off, BK)], rowv)
            pltpu.sync_copy(rowsb, p_sh.at[rowv], add=True)

        pltpu.sync_copy(col_hbm.at[pl.ds(toff, TAIL)], colv_t)
        colv_t[...] = colv_t[...] + coff
        pltpu.async_copy(y_hbm.at[colv_t], rowsb_t, sem).wait()
        pltpu.sync_copy(row_hbm.at[pl.ds(toff, TAIL)], rowv_t)
        pltpu.sync_copy(rowsb_t, p_sh.at[rowv_t], add=True)
        plsc.subcore_barrier()

        pltpu.sync_copy(p_sh.at[pl.ds(nbase, SP)],
                        p_out.at[pl.ds(coff + nbase, SP)])

    return k(x_stack, rowi, coli)


def _tc_linear(p_stack, dis, w_stack, bias2d):
    BM = 1000

    def mm(p_ref, d_ref, w_ref, b_ref, o_ref):
        d = d_ref[:, 0:1]
        acc = jnp.dot(p_ref[0] * d, w_ref[0],
                      preferred_element_type=jnp.float32)
        acc = acc + jnp.dot(p_ref[1] * d, w_ref[1],
                            preferred_element_type=jnp.float32)
        o_ref[...] = jnp.maximum(acc + b_ref[...], 0.0)

    return pl.pallas_call(
        mm,
        grid=(N // BM,),
        in_specs=[
            pl.BlockSpec((NC, BM, H), lambda i: (0, i, 0)),
            pl.BlockSpec((BM, L), lambda i: (i, 0)),
            pl.BlockSpec((NC, H, D), lambda i: (0, 0, 0)),
            pl.BlockSpec((1, D), lambda i: (0, 0)),
        ],
        out_specs=pl.BlockSpec((BM, D), lambda i: (i, 0)),
        out_shape=jax.ShapeDtypeStruct((N, D), jnp.float32),
    )(p_stack, dis, w_stack, bias2d)


def kernel(features, edge_index, weight, bias):
    features = features.astype(jnp.float32)
    rowi = edge_index[0].astype(jnp.int32)
    coli = edge_index[1].astype(jnp.int32)
    x_stack = jnp.stack([features[:, :H], features[:, H:]])
    p_flat, dis, _ = _sc_propagate(x_stack, rowi, coli)
    p_stack = p_flat.reshape(NC, N, H)
    w_stack = weight.astype(jnp.float32).reshape(NC, H, D)
    bias2d = bias.astype(jnp.float32).reshape(1, D)
    return _tc_linear(p_stack, dis, w_stack, bias2d)


# speedup vs baseline: 4.8228x; 4.8228x over previous
import functools

import jax
import jax.numpy as jnp
from jax import lax
from jax.experimental import pallas as pl
from jax.experimental.pallas import tpu as pltpu
from jax.experimental.pallas import tpu_sc as plsc

N = 10000
NP = 10240
E = 160000
D = 256
NQ = 4
HQ = D // NQ
NC = 2
NS = 16
L = 16
EPT = E // NS
SP = NP // NS
CH = 128
NCH = SP // CH
BK = 80
NB = EPT // BK


def _sc_propagate(x_stack, rowi, coli):
    mesh = plsc.VectorSubcoreMesh(
        core_axis_name="c", subcore_axis_name="s", num_cores=NC, num_subcores=NS
    )

    @functools.partial(
        pl.kernel,
        out_type=(
            jax.ShapeDtypeStruct((NQ * NP, HQ), jnp.float32),
            jax.ShapeDtypeStruct((NP, L), jnp.float32),
            jax.ShapeDtypeStruct((NQ * NP, HQ), jnp.float32),
        ),
        mesh=mesh,
        compiler_params=pltpu.CompilerParams(use_tc_tiling_on_sc=False),
        scratch_types=(
            pltpu.VMEM_SHARED((NP, HQ), jnp.float32),
            pltpu.VMEM_SHARED((NP, L), jnp.float32),
            pltpu.VMEM((BK, L), jnp.float32),
            pltpu.VMEM((BK,), jnp.int32),
            pltpu.VMEM((BK,), jnp.int32),
            pltpu.VMEM((BK, HQ), jnp.float32),
            pltpu.VMEM((SP, L), jnp.float32),
            pltpu.VMEM((CH, HQ), jnp.float32),
            pltpu.VMEM((CH, HQ), jnp.float32),
            pltpu.VMEM((CH, L), jnp.float32),
            pltpu.SemaphoreType.DMA,
        ),
    )
    def k(x_hbm, row_hbm, col_hbm, p_out, dis_out, y_hbm,
          p_sh, deg_sh, ones_b, rowv, colv, rowsb, disv, xbuf, zbuf, zdeg,
          sem):
        cid = lax.axis_index("c")
        sid = lax.axis_index("s")
        nbase = sid * SP
        ebase = sid * EPT

        zv = jnp.zeros((L,), jnp.float32)
        ov = jnp.ones((L,), jnp.float32)

        @pl.loop(0, CH)
        def _(r):
            zdeg[r, :] = zv
            for c in range(HQ // L):
                zbuf[r, pl.ds(c * L, L)] = zv

        @pl.loop(0, BK)
        def _(r):
            ones_b[r, :] = ov

        for j in range(NCH):
            pltpu.sync_copy(zdeg, deg_sh.at[pl.ds(nbase + j * CH, CH)])
        plsc.subcore_barrier()

        @pl.loop(0, NB)
        def _(b):
            off = pl.multiple_of(ebase + b * BK, 8)
            pltpu.sync_copy(row_hbm.at[pl.ds(off, BK)], rowv)
            pltpu.sync_copy(ones_b, deg_sh.at[rowv], add=True)

        plsc.subcore_barrier()

        pltpu.sync_copy(deg_sh.at[pl.ds(nbase, SP)], disv)

        @pl.loop(0, SP)
        def _(r):
            x = disv[r, :]
            i = lax.bitcast_convert_type(x, jnp.int32)
            y = lax.bitcast_convert_type(
                jnp.int32(0x5F3759DF) - (i >> 1), jnp.float32)
            for _ in range(3):
                y = y * (1.5 - 0.5 * x * y * y)
            disv[r, :] = jnp.where(x > 0.5, y, 0.0)

        @pl.when(cid == 0)
        def _():
            pltpu.sync_copy(disv, dis_out.at[pl.ds(nbase, SP)])

        for q in range(NQ // NC):
            qidx = cid * (NQ // NC) + q
            qoff = qidx * NP

            for j in range(NCH):
                pltpu.sync_copy(zbuf, p_sh.at[pl.ds(nbase + j * CH, CH)])

            for j in range(NCH):
                r0 = nbase + j * CH
                pltpu.sync_copy(x_hbm.at[qidx, pl.ds(r0, CH)], xbuf)

                @pl.loop(0, CH)
                def _(rr):
                    dv = disv[j * CH + rr, :]
                    bc = lax.broadcast(dv[0], (L,))
                    for c in range(HQ // L):
                        xbuf[rr, pl.ds(c * L, L)] = (
                            xbuf[rr, pl.ds(c * L, L)] * bc)

                pltpu.sync_copy(xbuf, y_hbm.at[pl.ds(qoff + r0, CH)])
            plsc.subcore_barrier()

            @pl.loop(0, NB)
            def _(b):
                off = pl.multiple_of(ebase + b * BK, 8)
                pltpu.sync_copy(col_hbm.at[pl.ds(off, BK)], colv)
                for c in range(BK // L):
                    colv[pl.ds(c * L, L)] = colv[pl.ds(c * L, L)] + qoff
                pltpu.async_copy(y_hbm.at[colv], rowsb, sem).wait()
                pltpu.sync_copy(row_hbm.at[pl.ds(off, BK)], rowv)
                pltpu.sync_copy(rowsb, p_sh.at[rowv], add=True)

            plsc.subcore_barrier()

            for j in range(NCH):
                pltpu.sync_copy(p_sh.at[pl.ds(nbase + j * CH, CH)], xbuf)
                pltpu.sync_copy(xbuf, p_out.at[pl.ds(qoff + nbase + j * CH,
                                                     CH)])

    return k(x_stack, rowi, coli)


def _tc_linear(p_stack, dis, w_stack, bias2d):
    BM = 1000

    def mm(p_ref, d_ref, w_ref, b_ref, o_ref):
        d = d_ref[:, 0:1]
        acc = jnp.dot(p_ref[0] * d, w_ref[0],
                      preferred_element_type=jnp.float32)
        for qq in range(1, NQ):
            acc = acc + jnp.dot(p_ref[qq] * d, w_ref[qq],
                                preferred_element_type=jnp.float32)
        o_ref[...] = jnp.maximum(acc + b_ref[...], 0.0)

    return pl.pallas_call(
        mm,
        grid=(N // BM,),
        in_specs=[
            pl.BlockSpec((NQ, BM, HQ), lambda i: (0, i, 0)),
            pl.BlockSpec((BM, L), lambda i: (i, 0)),
            pl.BlockSpec((NQ, HQ, D), lambda i: (0, 0, 0)),
            pl.BlockSpec((1, D), lambda i: (0, 0)),
        ],
        out_specs=pl.BlockSpec((BM, D), lambda i: (i, 0)),
        out_shape=jax.ShapeDtypeStruct((N, D), jnp.float32),
    )(p_stack, dis, w_stack, bias2d)


def kernel(features, edge_index, weight, bias):
    features = features.astype(jnp.float32)
    rowi = edge_index[0].astype(jnp.int32)
    coli = edge_index[1].astype(jnp.int32)
    xp = jnp.pad(features, ((0, NP - N), (0, 0)))
    x_stack = xp.reshape(NP, NQ, HQ).transpose(1, 0, 2)
    p_flat, dis, _ = _sc_propagate(x_stack, rowi, coli)
    p_stack = p_flat.reshape(NQ, NP, HQ)[:, :N, :]
    dis = dis[:N]
    w_stack = weight.astype(jnp.float32).reshape(NQ, HQ, D)
    bias2d = bias.astype(jnp.float32).reshape(1, D)
    return _tc_linear(p_stack, dis, w_stack, bias2d)


# speedup vs baseline: 9.7676x; 2.0253x over previous
import functools

import jax
import jax.numpy as jnp
from jax import lax
from jax.experimental import pallas as pl
from jax.experimental.pallas import tpu as pltpu
from jax.experimental.pallas import tpu_sc as plsc

N = 10000
NP = 10240
E = 160000
D = 256
NQ = 4
HQ = D // NQ
NC = 2
NS = 16
L = 16
EPT = E // NS
SP = NP // NS
CH = 128
NCH = SP // CH
BK = 80
NB = EPT // BK


def _sc_propagate(x_stack, row2, col2):
    mesh = plsc.VectorSubcoreMesh(
        core_axis_name="c", subcore_axis_name="s", num_cores=NC, num_subcores=NS
    )

    @functools.partial(
        pl.kernel,
        out_type=(
            jax.ShapeDtypeStruct((NQ * NP, HQ), jnp.float32),
            jax.ShapeDtypeStruct((NP, L), jnp.float32),
            jax.ShapeDtypeStruct((NQ * NP, HQ), jnp.float32),
        ),
        mesh=mesh,
        compiler_params=pltpu.CompilerParams(use_tc_tiling_on_sc=False),
        scratch_types=(
            pltpu.VMEM_SHARED((NP, HQ), jnp.float32),
            pltpu.VMEM_SHARED((NP, L), jnp.float32),
            pltpu.VMEM((BK, L), jnp.float32),
            pltpu.VMEM((NB, BK), jnp.int32),
            pltpu.VMEM((NB, BK), jnp.int32),
            pltpu.VMEM((BK, HQ), jnp.float32),
            pltpu.VMEM((BK, HQ), jnp.float32),
            pltpu.VMEM((SP, L), jnp.float32),
            pltpu.VMEM((CH, HQ), jnp.float32),
            pltpu.VMEM((CH, L), jnp.float32),
            pltpu.SemaphoreType.DMA,
            pltpu.SemaphoreType.DMA,
            pltpu.SemaphoreType.DMA,
            pltpu.SemaphoreType.DMA,
        ),
    )
    def k(x_hbm, row2_hbm, col2_hbm, p_out, dis_out, y_hbm,
          p_sh, deg_sh, ones_b, rowall, colall, rowsb0, rowsb1, disv, xbuf,
          zdeg, gsem0, gsem1, ssem0, ssem1):
        cid = lax.axis_index("c")
        sid = lax.axis_index("s")
        nbase = sid * SP

        zv = jnp.zeros((L,), jnp.float32)
        ov = jnp.ones((L,), jnp.float32)

        pltpu.sync_copy(row2_hbm.at[sid], rowall)
        pltpu.sync_copy(col2_hbm.at[sid], colall)

        @pl.loop(0, CH)
        def _(r):
            zdeg[r, :] = zv
            for c in range(HQ // L):
                xbuf[r, pl.ds(c * L, L)] = zv

        @pl.loop(0, BK)
        def _(r):
            ones_b[r, :] = ov

        for j in range(NCH):
            pltpu.sync_copy(zdeg, deg_sh.at[pl.ds(nbase + j * CH, CH)])
        plsc.subcore_barrier()

        @pl.loop(0, NB // 5)
        def _(t):
            descs = [
                pltpu.async_copy(ones_b, deg_sh.at[rowall.at[t * 5 + i]],
                                 gsem0, add=True)
                for i in range(5)
            ]
            for dsc in descs:
                dsc.wait()

        plsc.subcore_barrier()

        pltpu.sync_copy(deg_sh.at[pl.ds(nbase, SP)], disv)

        @pl.loop(0, SP)
        def _(r):
            x = disv[r, :]
            i = lax.bitcast_convert_type(x, jnp.int32)
            y = lax.bitcast_convert_type(
                jnp.int32(0x5F3759DF) - (i >> 1), jnp.float32)
            for _ in range(3):
                y = y * (1.5 - 0.5 * x * y * y)
            disv[r, :] = jnp.where(x > 0.5, y, 0.0)

        @pl.when(cid == 0)
        def _():
            pltpu.sync_copy(disv, dis_out.at[pl.ds(nbase, SP)])

        for q in range(NQ // NC):
            qidx = cid * (NQ // NC) + q
            qoff = qidx * NP

            if q == 0:
                addv = lax.broadcast(qoff, (L,)).astype(jnp.int32)
            else:
                addv = jnp.full((L,), NP, jnp.int32)

            @pl.loop(0, NB)
            def _(r):
                for c in range(BK // L):
                    colall[r, pl.ds(c * L, L)] = (
                        colall[r, pl.ds(c * L, L)] + addv)

            @pl.loop(0, CH)
            def _(r):
                for c in range(HQ // L):
                    xbuf[r, pl.ds(c * L, L)] = zv

            for j in range(NCH):
                pltpu.sync_copy(xbuf, p_sh.at[pl.ds(nbase + j * CH, CH)])

            for j in range(NCH):
                r0 = nbase + j * CH
                pltpu.sync_copy(x_hbm.at[qidx, pl.ds(r0, CH)], xbuf)

                @pl.loop(0, CH)
                def _(rr):
                    dv = disv[j * CH + rr, :]
                    bc = lax.broadcast(dv[0], (L,))
                    for c in range(HQ // L):
                        xbuf[rr, pl.ds(c * L, L)] = (
                            xbuf[rr, pl.ds(c * L, L)] * bc)

                pltpu.sync_copy(xbuf, y_hbm.at[pl.ds(qoff + r0, CH)])
            plsc.subcore_barrier()

            @pl.loop(0, NB // 2)
            def _(t):
                b0 = t * 2
                b1 = t * 2 + 1
                g0 = pltpu.async_copy(y_hbm.at[colall.at[b0]], rowsb0, gsem0)
                g1 = pltpu.async_copy(y_hbm.at[colall.at[b1]], rowsb1, gsem1)
                g0.wait()
                s0 = pltpu.async_copy(rowsb0, p_sh.at[rowall.at[b0]],
                                      ssem0, add=True)
                g1.wait()
                s1 = pltpu.async_copy(rowsb1, p_sh.at[rowall.at[b1]],
                                      ssem1, add=True)
                s0.wait()
                s1.wait()

            gt = pltpu.async_copy(y_hbm.at[colall.at[NB - 1]], rowsb0, gsem0)
            gt.wait()
            st = pltpu.async_copy(rowsb0, p_sh.at[rowall.at[NB - 1]],
                                  ssem0, add=True)
            st.wait()
            plsc.subcore_barrier()

            for j in range(NCH):
                pltpu.sync_copy(p_sh.at[pl.ds(nbase + j * CH, CH)], xbuf)
                pltpu.sync_copy(xbuf, p_out.at[pl.ds(qoff + nbase + j * CH,
                                                     CH)])

    return k(x_stack, row2, col2)


def _tc_linear(p_stack, dis, w_stack, bias2d):
    BM = 1000

    def mm(p_ref, d_ref, w_ref, b_ref, o_ref):
        d = d_ref[:, 0:1]
        acc = jnp.dot(p_ref[0] * d, w_ref[0],
                      preferred_element_type=jnp.float32)
        for qq in range(1, NQ):
            acc = acc + jnp.dot(p_ref[qq] * d, w_ref[qq],
                                preferred_element_type=jnp.float32)
        o_ref[...] = jnp.maximum(acc + b_ref[...], 0.0)

    return pl.pallas_call(
        mm,
        grid=(N // BM,),
        in_specs=[
            pl.BlockSpec((NQ, BM, HQ), lambda i: (0, i, 0)),
            pl.BlockSpec((BM, L), lambda i: (i, 0)),
            pl.BlockSpec((NQ, HQ, D), lambda i: (0, 0, 0)),
            pl.BlockSpec((1, D), lambda i: (0, 0)),
        ],
        out_specs=pl.BlockSpec((BM, D), lambda i: (i, 0)),
        out_shape=jax.ShapeDtypeStruct((N, D), jnp.float32),
    )(p_stack, dis, w_stack, bias2d)


def kernel(features, edge_index, weight, bias):
    features = features.astype(jnp.float32)
    rowi = edge_index[0].astype(jnp.int32)
    coli = edge_index[1].astype(jnp.int32)
    row2 = rowi.reshape(NS, NB, BK)
    col2 = coli.reshape(NS, NB, BK)
    xp = jnp.pad(features, ((0, NP - N), (0, 0)))
    x_stack = xp.reshape(NP, NQ, HQ).transpose(1, 0, 2)
    p_flat, dis, _ = _sc_propagate(x_stack, row2, col2)
    p_stack = p_flat.reshape(NQ, NP, HQ)[:, :N, :]
    dis = dis[:N]
    w_stack = weight.astype(jnp.float32).reshape(NQ, HQ, D)
    bias2d = bias.astype(jnp.float32).reshape(1, D)
    return _tc_linear(p_stack, dis, w_stack, bias2d)
